# baseline (speedup 1.0000x reference)
import jax
import jax.numpy as jnp
from jax import lax
from jax.experimental import pallas as pl
from jax.experimental.pallas import tpu as pltpu
from jax.experimental.pallas import tpu_sc as plsc

NC, NS = 2, 16
NW = NC * NS
B, L = 16384, 20
R = B * L
CHUNK = 128
NBUF = 4
BT = B // CHUNK
QPW = BT // NW
PER_W = QPW * L
ROUNDS_PER_Q = L // NBUF

D_ID = 32
D_CAT = 16
D_OUT = D_ID + 3 * D_CAT
NT = D_OUT // 8
TILE = 8 * CHUNK
OUT_FLAT = L * NT * BT * TILE


def _emb_body(nid, sec, reg, ven, id_t, sec_t, reg_t, ven_t, out,
              slab, idx_x, id_r, sec_r, reg_r, ven_r, stage,
              sem_i, sem_g, sem_w):
    wid = lax.axis_index("s") * NC + lax.axis_index("c")
    iota16 = lax.iota(jnp.int32, 16)

    def load_slabs(bt):
        rows = pl.ds(bt * CHUNK, CHUNK)
        pltpu.sync_copy(nid.at[rows, :], slab.at[0])
        pltpu.sync_copy(sec.at[rows, :], slab.at[1])
        pltpu.sync_copy(reg.at[rows, :], slab.at[2])
        pltpu.sync_copy(ven.at[rows, :], slab.at[3])

    def extract_indices():
        def per_l(l, carry):
            cols = jnp.full((16,), l, jnp.int32)
            for a in range(4):
                for m in range(8):
                    idx_x[a, l, pl.ds(m * 16, 16)] = plsc.load_gather(
                        slab.at[a], [m * 16 + iota16, cols])
            return carry

        lax.fori_loop(0, L, per_l, 0)

    def gather_copies(l, b):
        return [
            pltpu.make_async_copy(id_t.at[idx_x.at[0, l]], id_r.at[b],
                                  sem_g.at[b]),
            pltpu.make_async_copy(sec_t.at[idx_x.at[1, l]], sec_r.at[b],
                                  sem_g.at[b]),
            pltpu.make_async_copy(reg_t.at[idx_x.at[2, l]], reg_r.at[b],
                                  sem_g.at[b]),
            pltpu.make_async_copy(ven_t.at[idx_x.at[3, l]], ven_r.at[b],
                                  sem_g.at[b]),
        ]

    def tile_src(b, t):
        if t < 4:
            return id_r.at[b], t * 8
        if t < 6:
            return sec_r.at[b], (t - 4) * 8
        if t < 8:
            return reg_r.at[b], (t - 6) * 8
        return ven_r.at[b], (t - 8) * 8

    def transpose(b):
        def jblk(j, carry):
            rows = j * 16 + iota16
            base = j * 16
            for t in range(NT):
                src, d0 = tile_src(b, t)
                for dr in range(8):
                    stage[b, t, pl.ds(dr * CHUNK + base, 16)] = (
                        plsc.load_gather(
                            src, [rows, jnp.full((16,), d0 + dr, jnp.int32)]))
            return carry

        lax.fori_loop(0, 8, jblk, 0)

    def write_copies(l, bt, b):
        return [
            pltpu.make_async_copy(
                stage.at[b, t],
                out.at[pl.ds(((l * NT + t) * BT + bt) * TILE, TILE)],
                sem_w.at[b])
            for t in range(NT)
        ]

    def outer(g, carry):
        q = g // ROUNDS_PER_Q
        l0 = (g % ROUNDS_PER_Q) * NBUF
        bt = wid * QPW + q

        @pl.when(g % ROUNDS_PER_Q == 0)
        def _():
            load_slabs(bt)
            extract_indices()

        for b in range(NBUF):
            @pl.when(g > 0)
            def _():
                for cp in write_copies(l0 + b, bt, b):
                    cp.wait()

        for b in range(NBUF):
            for cp in gather_copies(l0 + b, b):
                cp.start()

        for b in range(NBUF):
            for cp in gather_copies(l0 + b, b):
                cp.wait()
            transpose(b)
            for cp in write_copies(l0 + b, bt, b):
                cp.start()

        return carry

    lax.fori_loop(0, PER_W // NBUF, outer, 0, unroll=False)

    for b in range(NBUF):
        for cp in write_copies(0, wid * QPW, b):
            cp.wait()


def kernel(node_ids, cat_sector, cat_region, cat_venue,
           id_table, sector_table, region_table, venue_table):
    call = pl.kernel(
        _emb_body,
        out_type=jax.ShapeDtypeStruct((OUT_FLAT,), jnp.float32),
        mesh=plsc.VectorSubcoreMesh(
            core_axis_name="c", subcore_axis_name="s",
            num_cores=NC, num_subcores=NS),
        scratch_types=[
            pltpu.VMEM((4, CHUNK, L), jnp.int32),
            pltpu.VMEM((4, L, CHUNK), jnp.int32),
            pltpu.VMEM((NBUF, CHUNK, D_ID), jnp.float32),
            pltpu.VMEM((NBUF, CHUNK, D_CAT), jnp.float32),
            pltpu.VMEM((NBUF, CHUNK, D_CAT), jnp.float32),
            pltpu.VMEM((NBUF, CHUNK, D_CAT), jnp.float32),
            pltpu.VMEM((NBUF, NT, TILE), jnp.float32),
            pltpu.SemaphoreType.DMA((NBUF,)),
            pltpu.SemaphoreType.DMA((NBUF,)),
            pltpu.SemaphoreType.DMA((NBUF,)),
        ],
        compiler_params=pltpu.CompilerParams(use_tc_tiling_on_sc=False,
                                             needs_layout_passes=False),
    )
    flat = call(node_ids.astype(jnp.int32), cat_sector.astype(jnp.int32),
                cat_region.astype(jnp.int32), cat_venue.astype(jnp.int32),
                id_table, sector_table, region_table, venue_table)
    return (flat.reshape(L, NT, BT, 8, CHUNK)
                .transpose(2, 4, 0, 1, 3)
                .reshape(B, L, D_OUT))


# speedup vs baseline: 1.0086x; 1.0086x over previous
import jax
import jax.numpy as jnp
from jax import lax
from jax.experimental import pallas as pl
from jax.experimental.pallas import tpu as pltpu
from jax.experimental.pallas import tpu_sc as plsc

NC, NS = 2, 16
NW = NC * NS
B, L = 16384, 20
R = B * L
CHUNK = 128
NBUF = 4
BT = B // CHUNK
QPW = BT // NW
PER_W = QPW * L
ROUNDS_PER_Q = L // NBUF

D_ID = 32
D_CAT = 16
D_OUT = D_ID + 3 * D_CAT
NT = D_OUT // 8
TILE = 8 * CHUNK
OUT_FLAT = L * NT * BT * TILE


def _emb_body(nid, sec, reg, ven, id_t, sec_t, reg_t, ven_t, out,
              slab, idx_x, id_r, sec_r, reg_r, ven_r, stage,
              sem_i, sem_g, sem_w):
    wid = lax.axis_index("s") * NC + lax.axis_index("c")
    iota16 = lax.iota(jnp.int32, 16)

    def load_slabs(bt):
        rows = pl.ds(bt * CHUNK * L, CHUNK * L)
        pltpu.sync_copy(nid.at[rows], slab.at[0])
        pltpu.sync_copy(sec.at[rows], slab.at[1])
        pltpu.sync_copy(reg.at[rows], slab.at[2])
        pltpu.sync_copy(ven.at[rows], slab.at[3])

    iota_l = iota16 * L

    def extract_indices():
        def per_l(l, carry):
            for a in range(4):
                for m in range(8):
                    idx_x[a, l, pl.ds(m * 16, 16)] = plsc.load_gather(
                        slab.at[a], [iota_l + (m * 16 * L + l)])
            return carry

        lax.fori_loop(0, L, per_l, 0)

    def gather_copies(l, b):
        return [
            pltpu.make_async_copy(id_t.at[idx_x.at[0, l]], id_r.at[b],
                                  sem_g.at[b]),
            pltpu.make_async_copy(sec_t.at[idx_x.at[1, l]], sec_r.at[b],
                                  sem_g.at[b]),
            pltpu.make_async_copy(reg_t.at[idx_x.at[2, l]], reg_r.at[b],
                                  sem_g.at[b]),
            pltpu.make_async_copy(ven_t.at[idx_x.at[3, l]], ven_r.at[b],
                                  sem_g.at[b]),
        ]

    def tile_src(b, t):
        if t < 4:
            return id_r.at[b], t * 8
        if t < 6:
            return sec_r.at[b], (t - 4) * 8
        if t < 8:
            return reg_r.at[b], (t - 6) * 8
        return ven_r.at[b], (t - 8) * 8

    def transpose(b):
        def jblk(j, carry):
            rows = j * 16 + iota16
            base = j * 16
            for t in range(NT):
                src, d0 = tile_src(b, t)
                for dr in range(8):
                    stage[b, t, pl.ds(dr * CHUNK + base, 16)] = (
                        plsc.load_gather(
                            src, [rows, jnp.full((16,), d0 + dr, jnp.int32)]))
            return carry

        lax.fori_loop(0, 8, jblk, 0)

    def write_copies(l, bt, b):
        return [
            pltpu.make_async_copy(
                stage.at[b, t],
                out.at[pl.ds(((l * NT + t) * BT + bt) * TILE, TILE)],
                sem_w.at[b])
            for t in range(NT)
        ]

    def outer(g, carry):
        q = g // ROUNDS_PER_Q
        l0 = (g % ROUNDS_PER_Q) * NBUF
        bt = wid * QPW + q

        @pl.when(g % ROUNDS_PER_Q == 0)
        def _():
            load_slabs(bt)
            extract_indices()

        for b in range(NBUF):
            @pl.when(g > 0)
            def _():
                for cp in write_copies(l0 + b, bt, b):
                    cp.wait()

        for b in range(NBUF):
            for cp in gather_copies(l0 + b, b):
                cp.start()

        for b in range(NBUF):
            for cp in gather_copies(l0 + b, b):
                cp.wait()
            transpose(b)
            for cp in write_copies(l0 + b, bt, b):
                cp.start()

        return carry

    lax.fori_loop(0, PER_W // NBUF, outer, 0, unroll=False)

    for b in range(NBUF):
        for cp in write_copies(0, wid * QPW, b):
            cp.wait()


def kernel(node_ids, cat_sector, cat_region, cat_venue,
           id_table, sector_table, region_table, venue_table):
    call = pl.kernel(
        _emb_body,
        out_type=jax.ShapeDtypeStruct((OUT_FLAT,), jnp.float32),
        mesh=plsc.VectorSubcoreMesh(
            core_axis_name="c", subcore_axis_name="s",
            num_cores=NC, num_subcores=NS),
        scratch_types=[
            pltpu.VMEM((4, CHUNK * L), jnp.int32),
            pltpu.VMEM((4, L, CHUNK), jnp.int32),
            pltpu.VMEM((NBUF, CHUNK, D_ID), jnp.float32),
            pltpu.VMEM((NBUF, CHUNK, D_CAT), jnp.float32),
            pltpu.VMEM((NBUF, CHUNK, D_CAT), jnp.float32),
            pltpu.VMEM((NBUF, CHUNK, D_CAT), jnp.float32),
            pltpu.VMEM((NBUF, NT, TILE), jnp.float32),
            pltpu.SemaphoreType.DMA((NBUF,)),
            pltpu.SemaphoreType.DMA((NBUF,)),
            pltpu.SemaphoreType.DMA((NBUF,)),
        ],
        compiler_params=pltpu.CompilerParams(use_tc_tiling_on_sc=False,
                                             needs_layout_passes=False),
    )
    flat = call(node_ids.reshape(-1).astype(jnp.int32),
                cat_sector.reshape(-1).astype(jnp.int32),
                cat_region.reshape(-1).astype(jnp.int32),
                cat_venue.reshape(-1).astype(jnp.int32),
                id_table, sector_table, region_table, venue_table)
    return (flat.reshape(L, NT, BT, 8, CHUNK)
                .transpose(2, 4, 0, 1, 3)
                .reshape(B, L, D_OUT))
